# baseline (speedup 1.0000x reference)
import functools

import jax
import jax.numpy as jnp
from jax import lax
from jax.experimental import pallas as pl
from jax.experimental.pallas import tpu as pltpu
from jax.experimental.pallas import tpu_sc as plsc

_N = 10000
_E = 320000
_D = 128
_G = 128
_NB = 10
_BN = _N // _NB

_K = 128
_NC = 2
_NS = 16
_NW = _NC * _NS
_CPT = 80
_EP = _NW * _CPT * _K
_NP = 10240
_RPT = _NP // _NS


def _sc_segment_sum(h, src1, dst1, zeros):
    mesh = plsc.VectorSubcoreMesh(core_axis_name="c", subcore_axis_name="s")

    @functools.partial(
        pl.kernel,
        out_type=jax.ShapeDtypeStruct((_NC, _NP, _D), jnp.float32),
        mesh=mesh,
        scratch_types=[
            pltpu.VMEM((2, _K), jnp.int32),
            pltpu.VMEM((2, _K), jnp.int32),
            pltpu.VMEM((2, _K, _D), jnp.float32),
            pltpu.VMEM_SHARED((_NP, _D), jnp.float32),
            pltpu.SemaphoreType.DMA,
            pltpu.SemaphoreType.DMA,
            pltpu.SemaphoreType.DMA,
            pltpu.SemaphoreType.DMA,
        ],
    )
    def k(h_hbm, src_hbm, dst_hbm, z_hbm, out_hbm, sidx, didx, rows, agg,
          semi0, semi1, semg0, semg1):
        c = lax.axis_index("c")
        s = lax.axis_index("s")
        w = s * _NC + c
        base0 = w * _CPT
        semi = (semi0, semi1)
        semg = (semg0, semg1)

        def eoff(chunk):
            return pl.multiple_of(chunk * _K, _K)

        def issue_idx(chunk, b):
            pltpu.async_copy(src_hbm.at[pl.ds(eoff(chunk), _K)],
                             sidx.at[b], semi[b])
            pltpu.async_copy(dst_hbm.at[pl.ds(eoff(chunk), _K)],
                             didx.at[b], semi[b])

        def wait_idx(chunk, b):
            pltpu.make_async_copy(src_hbm.at[pl.ds(eoff(chunk), _K)],
                                  sidx.at[b], semi[b]).wait()
            pltpu.make_async_copy(dst_hbm.at[pl.ds(eoff(chunk), _K)],
                                  didx.at[b], semi[b]).wait()

        def issue_gather(b):
            pltpu.async_copy(h_hbm.at[sidx.at[b]], rows.at[b], semg[b])

        def wait_gather(b):
            pltpu.make_async_copy(h_hbm.at[sidx.at[b]], rows.at[b],
                                  semg[b]).wait()

        issue_idx(base0, 0)
        issue_idx(base0 + 1, 1)
        pltpu.sync_copy(z_hbm.at[pl.ds(s * _RPT, _RPT)],
                        agg.at[pl.ds(s * _RPT, _RPT)])
        plsc.subcore_barrier()

        def step(j, carry):
            for b in range(2):
                jj = j * 2 + b
                chunk = base0 + jj
                wait_idx(chunk, b)
                issue_gather(b)
                wait_gather(b)
                pltpu.sync_copy(rows.at[b], agg.at[didx.at[b]], add=True)

                @pl.when(jj + 2 < _CPT)
                def _():
                    issue_idx(chunk + 2, b)
            return carry

        lax.fori_loop(0, _CPT // 2, step, 0)

        plsc.subcore_barrier()
        pltpu.sync_copy(agg.at[pl.ds(s * _RPT, _RPT)],
                        out_hbm.at[c].at[pl.ds(s * _RPT, _RPT)])

    return k(h, src1, dst1, zeros)


def _mlp_stats_body(h_ref, a0_ref, a1_ref, w1_ref, b1_ref, w2_ref, b2_ref,
                    t_ref, st_ref):
    i = pl.program_id(0)
    m = h_ref[...] + a0_ref[...] + a1_ref[...]
    z = jnp.dot(m, w1_ref[...], preferred_element_type=jnp.float32)
    z = jnp.maximum(z + b1_ref[...], 0.0)
    t = jnp.dot(z, w2_ref[...], preferred_element_type=jnp.float32)
    t = jnp.maximum(t + b2_ref[...], 0.0)
    t_ref[...] = t
    stats = jnp.concatenate([jnp.sum(t, 0, keepdims=True),
                             jnp.sum(t * t, 0, keepdims=True)], axis=0)

    @pl.when(i == 0)
    def _():
        st_ref[...] = stats

    @pl.when(i > 0)
    def _():
        st_ref[...] += stats


def _tc_mlp_stats(h, a0, a1, w1, b1, w2, b2):
    blk = lambda i: (i, 0)
    const = lambda i: (0, 0)
    return pl.pallas_call(
        _mlp_stats_body,
        grid=(_NB,),
        in_specs=[
            pl.BlockSpec((_BN, _D), blk),
            pl.BlockSpec((_BN, _D), blk),
            pl.BlockSpec((_BN, _D), blk),
            pl.BlockSpec((_D, _D), const),
            pl.BlockSpec((1, _D), const),
            pl.BlockSpec((_D, _D), const),
            pl.BlockSpec((1, _D), const),
        ],
        out_specs=[
            pl.BlockSpec((_BN, _D), blk),
            pl.BlockSpec((2, _D), const),
        ],
        out_shape=[
            jax.ShapeDtypeStruct((_N, _D), jnp.float32),
            jax.ShapeDtypeStruct((2, _D), jnp.float32),
        ],
    )(h, a0, a1, w1, b1, w2, b2)


def _norm_pool_body(t_ref, st_ref, g_ref, b_ref, bt_ref, h_ref, p_ref):
    i = pl.program_id(0)
    mean = st_ref[0:1, :] * (1.0 / _N)
    var = st_ref[1:2, :] * (1.0 / _N) - mean * mean
    scale = lax.rsqrt(var + 1e-5) * g_ref[...]
    off = b_ref[...] - mean * scale
    hh = t_ref[...] * scale + off
    h_ref[...] = hh
    bt = bt_ref[0, :, :]
    gids = lax.broadcasted_iota(jnp.int32, (_G, _BN), 0)
    onehot = (bt == gids).astype(jnp.float32)
    contrib = jnp.dot(onehot, hh, preferred_element_type=jnp.float32)

    @pl.when(i == 0)
    def _():
        p_ref[...] = contrib

    @pl.when(i > 0)
    def _():
        p_ref[...] += contrib


def _tc_norm_pool(t, stats, gamma, beta, batch3):
    blk = lambda i: (i, 0)
    const = lambda i: (0, 0)
    return pl.pallas_call(
        _norm_pool_body,
        grid=(_NB,),
        in_specs=[
            pl.BlockSpec((_BN, _D), blk),
            pl.BlockSpec((2, _D), const),
            pl.BlockSpec((1, _D), const),
            pl.BlockSpec((1, _D), const),
            pl.BlockSpec((1, 1, _BN), lambda i: (i, 0, 0)),
        ],
        out_specs=[
            pl.BlockSpec((_BN, _D), blk),
            pl.BlockSpec((_G, _D), const),
        ],
        out_shape=[
            jax.ShapeDtypeStruct((_N, _D), jnp.float32),
            jax.ShapeDtypeStruct((_G, _D), jnp.float32),
        ],
    )(t, stats, gamma, beta, batch3)


def _head_body(p0_ref, p1_ref, p2_ref, w1_ref, b1_ref, w2_ref, b2_ref,
               yn_ref, xn_ref):
    xc = jnp.concatenate([p0_ref[...], p1_ref[...], p2_ref[...]], axis=1)
    z = jnp.dot(xc, w1_ref[...], preferred_element_type=jnp.float32)
    z = jnp.maximum(z + b1_ref[...], 0.0)
    y = jnp.dot(z, w2_ref[...], preferred_element_type=jnp.float32) + b2_ref[...]
    xnorm = jnp.sqrt(jnp.sum(xc * xc, axis=1, keepdims=True))
    ynorm = jnp.sqrt(jnp.sum(y * y, axis=1, keepdims=True))
    xn_ref[...] = xc / jnp.maximum(xnorm, 1e-12)
    yn_ref[...] = y / jnp.maximum(ynorm, 1e-12)


def _tc_head(p0, p1, p2, pw1, pb1, pw2, pb2):
    H = 3 * _D
    return pl.pallas_call(
        _head_body,
        out_shape=[
            jax.ShapeDtypeStruct((_G, H), jnp.float32),
            jax.ShapeDtypeStruct((_G, H), jnp.float32),
        ],
    )(p0, p1, p2, pw1, pb1, pw2, pb2)


def kernel(x, edge_index, batch,
           l0_W1, l0_b1, l0_W2, l0_b2, l0_gamma, l0_beta,
           l1_W1, l1_b1, l1_W2, l1_b2, l1_gamma, l1_beta,
           l2_W1, l2_b1, l2_W2, l2_b2, l2_gamma, l2_beta,
           p_W1, p_b1, p_W2, p_b2):
    pad = _EP - _E
    src1 = jnp.concatenate([edge_index[0], jnp.zeros((pad,), jnp.int32)])
    dst1 = jnp.concatenate([edge_index[1], jnp.full((pad,), _N, jnp.int32)])
    batch3 = batch.reshape(_NB, 1, _BN)
    zeros = jnp.zeros((_NP, _D), jnp.float32)
    layers = [
        (l0_W1, l0_b1, l0_W2, l0_b2, l0_gamma, l0_beta),
        (l1_W1, l1_b1, l1_W2, l1_b2, l1_gamma, l1_beta),
        (l2_W1, l2_b1, l2_W2, l2_b2, l2_gamma, l2_beta),
    ]
    h = x
    pools = []
    for (w1, b1, w2, b2, g, b) in layers:
        a = _sc_segment_sum(h, src1, dst1, zeros)
        t, st = _tc_mlp_stats(h, a[0, :_N], a[1, :_N], w1, b1.reshape(1, _D),
                              w2, b2.reshape(1, _D))
        h, p = _tc_norm_pool(t, st, g.reshape(1, _D), b.reshape(1, _D), batch3)
        pools.append(p)
    yn, xn = _tc_head(pools[0], pools[1], pools[2],
                      p_W1, p_b1.reshape(1, 3 * _D), p_W2, p_b2.reshape(1, 3 * _D))
    return (yn, xn)


# speedup vs baseline: 1.0002x; 1.0002x over previous
import functools

import jax
import jax.numpy as jnp
from jax import lax
from jax.experimental import pallas as pl
from jax.experimental.pallas import tpu as pltpu
from jax.experimental.pallas import tpu_sc as plsc

_N = 10000
_E = 320000
_D = 128
_G = 128
_NB = 10
_BN = _N // _NB

_K = 128
_NC = 2
_NS = 16
_NW = _NC * _NS
_CPT = 80
_EP = _NW * _CPT * _K
_NP = 10240
_RPT = _NP // _NS


def _sc_segment_sum(h, src1, dst1, zeros):
    mesh = plsc.VectorSubcoreMesh(core_axis_name="c", subcore_axis_name="s")

    @functools.partial(
        pl.kernel,
        out_type=jax.ShapeDtypeStruct((_NC, _NP, _D), jnp.float32),
        mesh=mesh,
        scratch_types=[
            pltpu.VMEM((_K,), jnp.int32),
            pltpu.VMEM((_K,), jnp.int32),
            pltpu.VMEM((_K,), jnp.int32),
            pltpu.VMEM((_K,), jnp.int32),
            pltpu.VMEM((2, _K, _D), jnp.float32),
            pltpu.VMEM_SHARED((_NP, _D), jnp.float32),
            pltpu.SemaphoreType.DMA,
            pltpu.SemaphoreType.DMA,
            pltpu.SemaphoreType.DMA,
            pltpu.SemaphoreType.DMA,
        ],
    )
    def k(h_hbm, src_hbm, dst_hbm, z_hbm, out_hbm, sidx0, sidx1, didx0, didx1,
          rows, agg, semi0, semi1, semg0, semg1):
        c = lax.axis_index("c")
        s = lax.axis_index("s")
        w = s * _NC + c
        base0 = w * _CPT
        semi = (semi0, semi1)
        semg = (semg0, semg1)
        sidx = (sidx0, sidx1)
        didx = (didx0, didx1)

        def eoff(chunk):
            return pl.multiple_of(chunk * _K, _K)

        def issue_idx(chunk, b):
            pltpu.async_copy(src_hbm.at[pl.ds(eoff(chunk), _K)],
                             sidx[b], semi[b])
            pltpu.async_copy(dst_hbm.at[pl.ds(eoff(chunk), _K)],
                             didx[b], semi[b])

        def wait_idx(chunk, b):
            pltpu.make_async_copy(src_hbm.at[pl.ds(eoff(chunk), _K)],
                                  sidx[b], semi[b]).wait()
            pltpu.make_async_copy(dst_hbm.at[pl.ds(eoff(chunk), _K)],
                                  didx[b], semi[b]).wait()

        def issue_gather(b):
            pltpu.async_copy(h_hbm.at[sidx[b]], rows.at[b], semg[b])

        def wait_gather(b):
            pltpu.make_async_copy(h_hbm.at[sidx[b]], rows.at[b],
                                  semg[b]).wait()

        issue_idx(base0, 0)
        issue_idx(base0 + 1, 1)
        pltpu.sync_copy(z_hbm.at[pl.ds(s * _RPT, _RPT)],
                        agg.at[pl.ds(s * _RPT, _RPT)])
        plsc.subcore_barrier()

        def step(j, carry):
            for b in range(2):
                jj = j * 2 + b
                chunk = base0 + jj
                wait_idx(chunk, b)
                issue_gather(b)
                wait_gather(b)
                pltpu.sync_copy(rows.at[b], agg.at[didx[b]], add=True)

                @pl.when(jj + 2 < _CPT)
                def _():
                    issue_idx(chunk + 2, b)
            return carry

        lax.fori_loop(0, _CPT // 2, step, 0)

        plsc.subcore_barrier()
        pltpu.sync_copy(agg.at[pl.ds(s * _RPT, _RPT)],
                        out_hbm.at[c].at[pl.ds(s * _RPT, _RPT)])

    return k(h, src1, dst1, zeros)


def _mlp_stats_body(h_ref, a0_ref, a1_ref, w1_ref, b1_ref, w2_ref, b2_ref,
                    t_ref, st_ref):
    i = pl.program_id(0)
    m = h_ref[...] + a0_ref[...] + a1_ref[...]
    z = jnp.dot(m, w1_ref[...], preferred_element_type=jnp.float32)
    z = jnp.maximum(z + b1_ref[...], 0.0)
    t = jnp.dot(z, w2_ref[...], preferred_element_type=jnp.float32)
    t = jnp.maximum(t + b2_ref[...], 0.0)
    t_ref[...] = t
    stats = jnp.concatenate([jnp.sum(t, 0, keepdims=True),
                             jnp.sum(t * t, 0, keepdims=True)], axis=0)

    @pl.when(i == 0)
    def _():
        st_ref[...] = stats

    @pl.when(i > 0)
    def _():
        st_ref[...] += stats


def _tc_mlp_stats(h, a0, a1, w1, b1, w2, b2):
    blk = lambda i: (i, 0)
    const = lambda i: (0, 0)
    return pl.pallas_call(
        _mlp_stats_body,
        grid=(_NB,),
        in_specs=[
            pl.BlockSpec((_BN, _D), blk),
            pl.BlockSpec((_BN, _D), blk),
            pl.BlockSpec((_BN, _D), blk),
            pl.BlockSpec((_D, _D), const),
            pl.BlockSpec((1, _D), const),
            pl.BlockSpec((_D, _D), const),
            pl.BlockSpec((1, _D), const),
        ],
        out_specs=[
            pl.BlockSpec((_BN, _D), blk),
            pl.BlockSpec((2, _D), const),
        ],
        out_shape=[
            jax.ShapeDtypeStruct((_N, _D), jnp.float32),
            jax.ShapeDtypeStruct((2, _D), jnp.float32),
        ],
    )(h, a0, a1, w1, b1, w2, b2)


def _norm_pool_body(t_ref, st_ref, g_ref, b_ref, bt_ref, h_ref, p_ref):
    i = pl.program_id(0)
    mean = st_ref[0:1, :] * (1.0 / _N)
    var = st_ref[1:2, :] * (1.0 / _N) - mean * mean
    scale = lax.rsqrt(var + 1e-5) * g_ref[...]
    off = b_ref[...] - mean * scale
    hh = t_ref[...] * scale + off
    h_ref[...] = hh
    bt = bt_ref[0, :, :]
    gids = lax.broadcasted_iota(jnp.int32, (_G, _BN), 0)
    onehot = (bt == gids).astype(jnp.float32)
    contrib = jnp.dot(onehot, hh, preferred_element_type=jnp.float32)

    @pl.when(i == 0)
    def _():
        p_ref[...] = contrib

    @pl.when(i > 0)
    def _():
        p_ref[...] += contrib


def _tc_norm_pool(t, stats, gamma, beta, batch3):
    blk = lambda i: (i, 0)
    const = lambda i: (0, 0)
    return pl.pallas_call(
        _norm_pool_body,
        grid=(_NB,),
        in_specs=[
            pl.BlockSpec((_BN, _D), blk),
            pl.BlockSpec((2, _D), const),
            pl.BlockSpec((1, _D), const),
            pl.BlockSpec((1, _D), const),
            pl.BlockSpec((1, 1, _BN), lambda i: (i, 0, 0)),
        ],
        out_specs=[
            pl.BlockSpec((_BN, _D), blk),
            pl.BlockSpec((_G, _D), const),
        ],
        out_shape=[
            jax.ShapeDtypeStruct((_N, _D), jnp.float32),
            jax.ShapeDtypeStruct((_G, _D), jnp.float32),
        ],
    )(t, stats, gamma, beta, batch3)


def _head_body(p0_ref, p1_ref, p2_ref, w1_ref, b1_ref, w2_ref, b2_ref,
               yn_ref, xn_ref):
    xc = jnp.concatenate([p0_ref[...], p1_ref[...], p2_ref[...]], axis=1)
    z = jnp.dot(xc, w1_ref[...], preferred_element_type=jnp.float32)
    z = jnp.maximum(z + b1_ref[...], 0.0)
    y = jnp.dot(z, w2_ref[...], preferred_element_type=jnp.float32) + b2_ref[...]
    xnorm = jnp.sqrt(jnp.sum(xc * xc, axis=1, keepdims=True))
    ynorm = jnp.sqrt(jnp.sum(y * y, axis=1, keepdims=True))
    xn_ref[...] = xc / jnp.maximum(xnorm, 1e-12)
    yn_ref[...] = y / jnp.maximum(ynorm, 1e-12)


def _tc_head(p0, p1, p2, pw1, pb1, pw2, pb2):
    H = 3 * _D
    return pl.pallas_call(
        _head_body,
        out_shape=[
            jax.ShapeDtypeStruct((_G, H), jnp.float32),
            jax.ShapeDtypeStruct((_G, H), jnp.float32),
        ],
    )(p0, p1, p2, pw1, pb1, pw2, pb2)


def kernel(x, edge_index, batch,
           l0_W1, l0_b1, l0_W2, l0_b2, l0_gamma, l0_beta,
           l1_W1, l1_b1, l1_W2, l1_b2, l1_gamma, l1_beta,
           l2_W1, l2_b1, l2_W2, l2_b2, l2_gamma, l2_beta,
           p_W1, p_b1, p_W2, p_b2):
    pad = _EP - _E
    src1 = jnp.concatenate([edge_index[0], jnp.zeros((pad,), jnp.int32)])
    dst1 = jnp.concatenate([edge_index[1], jnp.full((pad,), _N, jnp.int32)])
    batch3 = batch.reshape(_NB, 1, _BN)
    zeros = jnp.zeros((_NP, _D), jnp.float32)
    layers = [
        (l0_W1, l0_b1, l0_W2, l0_b2, l0_gamma, l0_beta),
        (l1_W1, l1_b1, l1_W2, l1_b2, l1_gamma, l1_beta),
        (l2_W1, l2_b1, l2_W2, l2_b2, l2_gamma, l2_beta),
    ]
    h = x
    pools = []
    for (w1, b1, w2, b2, g, b) in layers:
        a = _sc_segment_sum(h, src1, dst1, zeros)
        t, st = _tc_mlp_stats(h, a[0, :_N], a[1, :_N], w1, b1.reshape(1, _D),
                              w2, b2.reshape(1, _D))
        h, p = _tc_norm_pool(t, st, g.reshape(1, _D), b.reshape(1, _D), batch3)
        pools.append(p)
    yn, xn = _tc_head(pools[0], pools[1], pools[2],
                      p_W1, p_b1.reshape(1, 3 * _D), p_W2, p_b2.reshape(1, 3 * _D))
    return (yn, xn)


# speedup vs baseline: 1.0790x; 1.0787x over previous
import functools

import jax
import jax.numpy as jnp
from jax import lax
from jax.experimental import pallas as pl
from jax.experimental.pallas import tpu as pltpu
from jax.experimental.pallas import tpu_sc as plsc

_N = 10000
_E = 320000
_D = 128
_G = 128
_NB = 10
_BN = _N // _NB

_K = 128
_NC = 2
_NS = 16
_NW = _NC * _NS
_CPT = 80
_EP = _NW * _CPT * _K
_NP = 10240
_RPT = _NP // _NS


def _sc_segment_sum(h, src1, dst1, zeros):
    mesh = plsc.VectorSubcoreMesh(core_axis_name="c", subcore_axis_name="s")

    @functools.partial(
        pl.kernel,
        out_type=jax.ShapeDtypeStruct((_NC, _NP, _D), jnp.float32),
        mesh=mesh,
        scratch_types=[
            pltpu.VMEM((_K,), jnp.int32),
            pltpu.VMEM((_K,), jnp.int32),
            pltpu.VMEM((_K,), jnp.int32),
            pltpu.VMEM((_K,), jnp.int32),
            pltpu.VMEM((2, _K, _D), jnp.float32),
            pltpu.VMEM_SHARED((_NP, _D), jnp.float32),
            pltpu.SemaphoreType.DMA,
            pltpu.SemaphoreType.DMA,
            pltpu.SemaphoreType.DMA,
            pltpu.SemaphoreType.DMA,
        ],
    )
    def k(h_hbm, src_hbm, dst_hbm, z_hbm, out_hbm, sidx0, sidx1, didx0, didx1,
          rows, agg, semi0, semi1, semg0, semg1):
        c = lax.axis_index("c")
        s = lax.axis_index("s")
        w = s * _NC + c
        base0 = w * _CPT
        semi = (semi0, semi1)
        semg = (semg0, semg1)
        sidx = (sidx0, sidx1)
        didx = (didx0, didx1)

        def eoff(chunk):
            return pl.multiple_of(chunk * _K, _K)

        def issue_idx(chunk, b):
            pltpu.async_copy(src_hbm.at[pl.ds(eoff(chunk), _K)],
                             sidx[b], semi[b])
            pltpu.async_copy(dst_hbm.at[pl.ds(eoff(chunk), _K)],
                             didx[b], semi[b])

        def wait_idx(chunk, b):
            pltpu.make_async_copy(src_hbm.at[pl.ds(eoff(chunk), _K)],
                                  sidx[b], semi[b]).wait()
            pltpu.make_async_copy(dst_hbm.at[pl.ds(eoff(chunk), _K)],
                                  didx[b], semi[b]).wait()

        def issue_gather(b):
            pltpu.async_copy(h_hbm.at[sidx[b]], rows.at[b], semg[b])

        def wait_gather(b):
            pltpu.make_async_copy(h_hbm.at[sidx[b]], rows.at[b],
                                  semg[b]).wait()

        issue_idx(base0, 0)
        issue_idx(base0 + 1, 1)
        pltpu.sync_copy(z_hbm.at[pl.ds(s * _RPT, _RPT)],
                        agg.at[pl.ds(s * _RPT, _RPT)])
        plsc.subcore_barrier()

        wait_idx(base0, 0)
        issue_gather(0)

        def step(j, carry):
            for b in range(2):
                jj = j * 2 + b
                chunk = base0 + jj
                wait_gather(b)

                @pl.when(jj + 1 < _CPT)
                def _():
                    wait_idx(chunk + 1, 1 - b)
                    issue_gather(1 - b)

                pltpu.sync_copy(rows.at[b], agg.at[didx[b]], add=True)

                @pl.when(jj + 2 < _CPT)
                def _():
                    issue_idx(chunk + 2, b)
            return carry

        lax.fori_loop(0, _CPT // 2, step, 0)

        plsc.subcore_barrier()
        pltpu.sync_copy(agg.at[pl.ds(s * _RPT, _RPT)],
                        out_hbm.at[c].at[pl.ds(s * _RPT, _RPT)])

    return k(h, src1, dst1, zeros)


def _mlp_stats_body(h_ref, a0_ref, a1_ref, w1_ref, b1_ref, w2_ref, b2_ref,
                    t_ref, st_ref):
    i = pl.program_id(0)
    m = h_ref[...] + a0_ref[...] + a1_ref[...]
    z = jnp.dot(m, w1_ref[...], preferred_element_type=jnp.float32)
    z = jnp.maximum(z + b1_ref[...], 0.0)
    t = jnp.dot(z, w2_ref[...], preferred_element_type=jnp.float32)
    t = jnp.maximum(t + b2_ref[...], 0.0)
    t_ref[...] = t
    stats = jnp.concatenate([jnp.sum(t, 0, keepdims=True),
                             jnp.sum(t * t, 0, keepdims=True)], axis=0)

    @pl.when(i == 0)
    def _():
        st_ref[...] = stats

    @pl.when(i > 0)
    def _():
        st_ref[...] += stats


def _tc_mlp_stats(h, a0, a1, w1, b1, w2, b2):
    blk = lambda i: (i, 0)
    const = lambda i: (0, 0)
    return pl.pallas_call(
        _mlp_stats_body,
        grid=(_NB,),
        in_specs=[
            pl.BlockSpec((_BN, _D), blk),
            pl.BlockSpec((_BN, _D), blk),
            pl.BlockSpec((_BN, _D), blk),
            pl.BlockSpec((_D, _D), const),
            pl.BlockSpec((1, _D), const),
            pl.BlockSpec((_D, _D), const),
            pl.BlockSpec((1, _D), const),
        ],
        out_specs=[
            pl.BlockSpec((_BN, _D), blk),
            pl.BlockSpec((2, _D), const),
        ],
        out_shape=[
            jax.ShapeDtypeStruct((_N, _D), jnp.float32),
            jax.ShapeDtypeStruct((2, _D), jnp.float32),
        ],
    )(h, a0, a1, w1, b1, w2, b2)


def _norm_pool_body(t_ref, st_ref, g_ref, b_ref, bt_ref, h_ref, p_ref):
    i = pl.program_id(0)
    mean = st_ref[0:1, :] * (1.0 / _N)
    var = st_ref[1:2, :] * (1.0 / _N) - mean * mean
    scale = lax.rsqrt(var + 1e-5) * g_ref[...]
    off = b_ref[...] - mean * scale
    hh = t_ref[...] * scale + off
    h_ref[...] = hh
    bt = bt_ref[0, :, :]
    gids = lax.broadcasted_iota(jnp.int32, (_G, _BN), 0)
    onehot = (bt == gids).astype(jnp.float32)
    contrib = jnp.dot(onehot, hh, preferred_element_type=jnp.float32)

    @pl.when(i == 0)
    def _():
        p_ref[...] = contrib

    @pl.when(i > 0)
    def _():
        p_ref[...] += contrib


def _tc_norm_pool(t, stats, gamma, beta, batch3):
    blk = lambda i: (i, 0)
    const = lambda i: (0, 0)
    return pl.pallas_call(
        _norm_pool_body,
        grid=(_NB,),
        in_specs=[
            pl.BlockSpec((_BN, _D), blk),
            pl.BlockSpec((2, _D), const),
            pl.BlockSpec((1, _D), const),
            pl.BlockSpec((1, _D), const),
            pl.BlockSpec((1, 1, _BN), lambda i: (i, 0, 0)),
        ],
        out_specs=[
            pl.BlockSpec((_BN, _D), blk),
            pl.BlockSpec((_G, _D), const),
        ],
        out_shape=[
            jax.ShapeDtypeStruct((_N, _D), jnp.float32),
            jax.ShapeDtypeStruct((_G, _D), jnp.float32),
        ],
    )(t, stats, gamma, beta, batch3)


def _head_body(p0_ref, p1_ref, p2_ref, w1_ref, b1_ref, w2_ref, b2_ref,
               yn_ref, xn_ref):
    xc = jnp.concatenate([p0_ref[...], p1_ref[...], p2_ref[...]], axis=1)
    z = jnp.dot(xc, w1_ref[...], preferred_element_type=jnp.float32)
    z = jnp.maximum(z + b1_ref[...], 0.0)
    y = jnp.dot(z, w2_ref[...], preferred_element_type=jnp.float32) + b2_ref[...]
    xnorm = jnp.sqrt(jnp.sum(xc * xc, axis=1, keepdims=True))
    ynorm = jnp.sqrt(jnp.sum(y * y, axis=1, keepdims=True))
    xn_ref[...] = xc / jnp.maximum(xnorm, 1e-12)
    yn_ref[...] = y / jnp.maximum(ynorm, 1e-12)


def _tc_head(p0, p1, p2, pw1, pb1, pw2, pb2):
    H = 3 * _D
    return pl.pallas_call(
        _head_body,
        out_shape=[
            jax.ShapeDtypeStruct((_G, H), jnp.float32),
            jax.ShapeDtypeStruct((_G, H), jnp.float32),
        ],
    )(p0, p1, p2, pw1, pb1, pw2, pb2)


def kernel(x, edge_index, batch,
           l0_W1, l0_b1, l0_W2, l0_b2, l0_gamma, l0_beta,
           l1_W1, l1_b1, l1_W2, l1_b2, l1_gamma, l1_beta,
           l2_W1, l2_b1, l2_W2, l2_b2, l2_gamma, l2_beta,
           p_W1, p_b1, p_W2, p_b2):
    pad = _EP - _E
    src1 = jnp.concatenate([edge_index[0], jnp.zeros((pad,), jnp.int32)])
    pad_dst = _N + (jnp.arange(pad, dtype=jnp.int32) % (_NP - _N))
    dst1 = jnp.concatenate([edge_index[1], pad_dst])
    batch3 = batch.reshape(_NB, 1, _BN)
    zeros = jnp.zeros((_NP, _D), jnp.float32)
    layers = [
        (l0_W1, l0_b1, l0_W2, l0_b2, l0_gamma, l0_beta),
        (l1_W1, l1_b1, l1_W2, l1_b2, l1_gamma, l1_beta),
        (l2_W1, l2_b1, l2_W2, l2_b2, l2_gamma, l2_beta),
    ]
    h = x
    pools = []
    for (w1, b1, w2, b2, g, b) in layers:
        a = _sc_segment_sum(h, src1, dst1, zeros)
        t, st = _tc_mlp_stats(h, a[0, :_N], a[1, :_N], w1, b1.reshape(1, _D),
                              w2, b2.reshape(1, _D))
        h, p = _tc_norm_pool(t, st, g.reshape(1, _D), b.reshape(1, _D), batch3)
        pools.append(p)
    yn, xn = _tc_head(pools[0], pools[1], pools[2],
                      p_W1, p_b1.reshape(1, 3 * _D), p_W2, p_b2.reshape(1, 3 * _D))
    return (yn, xn)


# speedup vs baseline: 1.0794x; 1.0004x over previous
import functools

import jax
import jax.numpy as jnp
from jax import lax
from jax.experimental import pallas as pl
from jax.experimental.pallas import tpu as pltpu
from jax.experimental.pallas import tpu_sc as plsc

_N = 10000
_E = 320000
_D = 128
_G = 128
_NB = 10
_BN = _N // _NB

_K = 128
_NC = 2
_NS = 16
_NW = _NC * _NS
_CPT = 80
_EP = _NW * _CPT * _K
_NP = 10240
_RPT = _NP // _NS


def _sc_segment_sum(h, src1, dst1, zeros):
    mesh = plsc.VectorSubcoreMesh(core_axis_name="c", subcore_axis_name="s")

    @functools.partial(
        pl.kernel,
        out_type=jax.ShapeDtypeStruct((_NC, _NP, _D), jnp.float32),
        mesh=mesh,
        scratch_types=[
            pltpu.VMEM((_K,), jnp.int32),
            pltpu.VMEM((_K,), jnp.int32),
            pltpu.VMEM((_K,), jnp.int32),
            pltpu.VMEM((_K,), jnp.int32),
            pltpu.VMEM((_K, _D), jnp.float32),
            pltpu.VMEM((_K, _D), jnp.float32),
            pltpu.VMEM_SHARED((_NP, _D), jnp.float32),
            pltpu.SemaphoreType.DMA,
            pltpu.SemaphoreType.DMA,
            pltpu.SemaphoreType.DMA,
            pltpu.SemaphoreType.DMA,
        ],
    )
    def k(h_hbm, src_hbm, dst_hbm, z_hbm, out_hbm, sidx0, sidx1, didx0, didx1,
          rows0, rows1, agg, semi0, semi1, semg0, semg1):
        c = lax.axis_index("c")
        s = lax.axis_index("s")
        w = s * _NC + c
        base0 = w * _CPT
        semi = (semi0, semi1)
        semg = (semg0, semg1)
        sidx = (sidx0, sidx1)
        didx = (didx0, didx1)
        rows = (rows0, rows1)

        def eoff(chunk):
            return pl.multiple_of(chunk * _K, _K)

        def issue_idx(chunk, b):
            pltpu.async_copy(src_hbm.at[pl.ds(eoff(chunk), _K)],
                             sidx[b], semi[b])
            pltpu.async_copy(dst_hbm.at[pl.ds(eoff(chunk), _K)],
                             didx[b], semi[b])

        def wait_idx(chunk, b):
            pltpu.make_async_copy(src_hbm.at[pl.ds(eoff(chunk), _K)],
                                  sidx[b], semi[b]).wait()
            pltpu.make_async_copy(dst_hbm.at[pl.ds(eoff(chunk), _K)],
                                  didx[b], semi[b]).wait()

        def issue_gather(b):
            pltpu.async_copy(h_hbm.at[sidx[b]], rows[b], semg[b])

        def wait_gather(b):
            pltpu.make_async_copy(h_hbm.at[sidx[b]], rows[b],
                                  semg[b]).wait()

        issue_idx(base0, 0)
        issue_idx(base0 + 1, 1)
        pltpu.sync_copy(z_hbm.at[pl.ds(s * _RPT, _RPT)],
                        agg.at[pl.ds(s * _RPT, _RPT)])
        plsc.subcore_barrier()

        wait_idx(base0, 0)
        issue_gather(0)

        def step(j, carry):
            for b in range(2):
                jj = j * 2 + b
                chunk = base0 + jj
                wait_gather(b)

                @pl.when(jj + 1 < _CPT)
                def _():
                    wait_idx(chunk + 1, 1 - b)
                    issue_gather(1 - b)

                pltpu.sync_copy(rows[b], agg.at[didx[b]], add=True)

                @pl.when(jj + 2 < _CPT)
                def _():
                    issue_idx(chunk + 2, b)
            return carry

        lax.fori_loop(0, _CPT // 2, step, 0)

        plsc.subcore_barrier()
        pltpu.sync_copy(agg.at[pl.ds(s * _RPT, _RPT)],
                        out_hbm.at[c].at[pl.ds(s * _RPT, _RPT)])

    return k(h, src1, dst1, zeros)


def _mlp_stats_body(h_ref, a0_ref, a1_ref, w1_ref, b1_ref, w2_ref, b2_ref,
                    t_ref, st_ref):
    i = pl.program_id(0)
    m = h_ref[...] + a0_ref[...] + a1_ref[...]
    z = jnp.dot(m, w1_ref[...], preferred_element_type=jnp.float32)
    z = jnp.maximum(z + b1_ref[...], 0.0)
    t = jnp.dot(z, w2_ref[...], preferred_element_type=jnp.float32)
    t = jnp.maximum(t + b2_ref[...], 0.0)
    t_ref[...] = t
    stats = jnp.concatenate([jnp.sum(t, 0, keepdims=True),
                             jnp.sum(t * t, 0, keepdims=True)], axis=0)

    @pl.when(i == 0)
    def _():
        st_ref[...] = stats

    @pl.when(i > 0)
    def _():
        st_ref[...] += stats


def _tc_mlp_stats(h, a0, a1, w1, b1, w2, b2):
    blk = lambda i: (i, 0)
    const = lambda i: (0, 0)
    return pl.pallas_call(
        _mlp_stats_body,
        grid=(_NB,),
        in_specs=[
            pl.BlockSpec((_BN, _D), blk),
            pl.BlockSpec((_BN, _D), blk),
            pl.BlockSpec((_BN, _D), blk),
            pl.BlockSpec((_D, _D), const),
            pl.BlockSpec((1, _D), const),
            pl.BlockSpec((_D, _D), const),
            pl.BlockSpec((1, _D), const),
        ],
        out_specs=[
            pl.BlockSpec((_BN, _D), blk),
            pl.BlockSpec((2, _D), const),
        ],
        out_shape=[
            jax.ShapeDtypeStruct((_N, _D), jnp.float32),
            jax.ShapeDtypeStruct((2, _D), jnp.float32),
        ],
    )(h, a0, a1, w1, b1, w2, b2)


def _norm_pool_body(t_ref, st_ref, g_ref, b_ref, bt_ref, h_ref, p_ref):
    i = pl.program_id(0)
    mean = st_ref[0:1, :] * (1.0 / _N)
    var = st_ref[1:2, :] * (1.0 / _N) - mean * mean
    scale = lax.rsqrt(var + 1e-5) * g_ref[...]
    off = b_ref[...] - mean * scale
    hh = t_ref[...] * scale + off
    h_ref[...] = hh
    bt = bt_ref[0, :, :]
    gids = lax.broadcasted_iota(jnp.int32, (_G, _BN), 0)
    onehot = (bt == gids).astype(jnp.float32)
    contrib = jnp.dot(onehot, hh, preferred_element_type=jnp.float32)

    @pl.when(i == 0)
    def _():
        p_ref[...] = contrib

    @pl.when(i > 0)
    def _():
        p_ref[...] += contrib


def _tc_norm_pool(t, stats, gamma, beta, batch3):
    blk = lambda i: (i, 0)
    const = lambda i: (0, 0)
    return pl.pallas_call(
        _norm_pool_body,
        grid=(_NB,),
        in_specs=[
            pl.BlockSpec((_BN, _D), blk),
            pl.BlockSpec((2, _D), const),
            pl.BlockSpec((1, _D), const),
            pl.BlockSpec((1, _D), const),
            pl.BlockSpec((1, 1, _BN), lambda i: (i, 0, 0)),
        ],
        out_specs=[
            pl.BlockSpec((_BN, _D), blk),
            pl.BlockSpec((_G, _D), const),
        ],
        out_shape=[
            jax.ShapeDtypeStruct((_N, _D), jnp.float32),
            jax.ShapeDtypeStruct((_G, _D), jnp.float32),
        ],
    )(t, stats, gamma, beta, batch3)


def _head_body(p0_ref, p1_ref, p2_ref, w1_ref, b1_ref, w2_ref, b2_ref,
               yn_ref, xn_ref):
    xc = jnp.concatenate([p0_ref[...], p1_ref[...], p2_ref[...]], axis=1)
    z = jnp.dot(xc, w1_ref[...], preferred_element_type=jnp.float32)
    z = jnp.maximum(z + b1_ref[...], 0.0)
    y = jnp.dot(z, w2_ref[...], preferred_element_type=jnp.float32) + b2_ref[...]
    xnorm = jnp.sqrt(jnp.sum(xc * xc, axis=1, keepdims=True))
    ynorm = jnp.sqrt(jnp.sum(y * y, axis=1, keepdims=True))
    xn_ref[...] = xc / jnp.maximum(xnorm, 1e-12)
    yn_ref[...] = y / jnp.maximum(ynorm, 1e-12)


def _tc_head(p0, p1, p2, pw1, pb1, pw2, pb2):
    H = 3 * _D
    return pl.pallas_call(
        _head_body,
        out_shape=[
            jax.ShapeDtypeStruct((_G, H), jnp.float32),
            jax.ShapeDtypeStruct((_G, H), jnp.float32),
        ],
    )(p0, p1, p2, pw1, pb1, pw2, pb2)


def kernel(x, edge_index, batch,
           l0_W1, l0_b1, l0_W2, l0_b2, l0_gamma, l0_beta,
           l1_W1, l1_b1, l1_W2, l1_b2, l1_gamma, l1_beta,
           l2_W1, l2_b1, l2_W2, l2_b2, l2_gamma, l2_beta,
           p_W1, p_b1, p_W2, p_b2):
    pad = _EP - _E
    src1 = jnp.concatenate([edge_index[0], jnp.zeros((pad,), jnp.int32)])
    pad_dst = _N + (jnp.arange(pad, dtype=jnp.int32) % (_NP - _N))
    dst1 = jnp.concatenate([edge_index[1], pad_dst])
    batch3 = batch.reshape(_NB, 1, _BN)
    zeros = jnp.zeros((_NP, _D), jnp.float32)
    layers = [
        (l0_W1, l0_b1, l0_W2, l0_b2, l0_gamma, l0_beta),
        (l1_W1, l1_b1, l1_W2, l1_b2, l1_gamma, l1_beta),
        (l2_W1, l2_b1, l2_W2, l2_b2, l2_gamma, l2_beta),
    ]
    h = x
    pools = []
    for (w1, b1, w2, b2, g, b) in layers:
        a = _sc_segment_sum(h, src1, dst1, zeros)
        t, st = _tc_mlp_stats(h, a[0, :_N], a[1, :_N], w1, b1.reshape(1, _D),
                              w2, b2.reshape(1, _D))
        h, p = _tc_norm_pool(t, st, g.reshape(1, _D), b.reshape(1, _D), batch3)
        pools.append(p)
    yn, xn = _tc_head(pools[0], pools[1], pools[2],
                      p_W1, p_b1.reshape(1, 3 * _D), p_W2, p_b2.reshape(1, 3 * _D))
    return (yn, xn)


# speedup vs baseline: 1.9258x; 1.7841x over previous
import functools

import jax
import jax.numpy as jnp
from jax import lax
from jax.experimental import pallas as pl
from jax.experimental.pallas import tpu as pltpu
from jax.experimental.pallas import tpu_sc as plsc

_N = 10000
_E = 320000
_D = 128
_G = 128
_NB = 10
_BN = _N // _NB

_K = 128
_NC = 2
_NS = 16
_NW = _NC * _NS
_CHUNKS = _E // _K
_FULL = _CHUNKS // _NW
_REM = _CHUNKS - _FULL * _NW
_NP = 10240
_RPT = _NP // _NS


def _sc_segment_sum(h, src1, dst1, zeros):
    mesh = plsc.VectorSubcoreMesh(core_axis_name="c", subcore_axis_name="s")

    @functools.partial(
        pl.kernel,
        out_type=jax.ShapeDtypeStruct((_NC, _NP, _D), jnp.float32),
        mesh=mesh,
        scratch_types=[
            pltpu.VMEM((_K,), jnp.int32),
            pltpu.VMEM((_K,), jnp.int32),
            pltpu.VMEM((_K, _D), jnp.float32),
            pltpu.VMEM_SHARED((_NP, _D), jnp.float32),
            pltpu.SemaphoreType.DMA,
        ],
    )
    def k(h_hbm, src_hbm, dst_hbm, z_hbm, out_hbm, sbuf, dbuf, rows, agg, sem):
        c = lax.axis_index("c")
        s = lax.axis_index("s")
        w = s * _NC + c

        pltpu.sync_copy(z_hbm.at[pl.ds(s * _RPT, _RPT)],
                        agg.at[pl.ds(s * _RPT, _RPT)])
        plsc.subcore_barrier()

        def do_chunk(chunk):
            base = pl.multiple_of(chunk * _K, _K)
            pltpu.sync_copy(src_hbm.at[pl.ds(base, _K)], sbuf)
            pltpu.sync_copy(dst_hbm.at[pl.ds(base, _K)], dbuf)
            pltpu.async_copy(h_hbm.at[sbuf], rows, sem).wait()
            pltpu.sync_copy(rows, agg.at[dbuf], add=True)

        def body(j, carry):
            do_chunk(w + j * _NW)
            return carry

        lax.fori_loop(0, _FULL, body, 0)

        @pl.when(w < _REM)
        def _():
            do_chunk(w + _FULL * _NW)

        plsc.subcore_barrier()
        pltpu.sync_copy(agg.at[pl.ds(s * _RPT, _RPT)],
                        out_hbm.at[c].at[pl.ds(s * _RPT, _RPT)])

    return k(h, src1, dst1, zeros)


def _mlp_stats_body(h_ref, a0_ref, a1_ref, w1_ref, b1_ref, w2_ref, b2_ref,
                    t_ref, st_ref):
    i = pl.program_id(0)
    m = h_ref[...] + a0_ref[...] + a1_ref[...]
    z = jnp.dot(m, w1_ref[...], preferred_element_type=jnp.float32)
    z = jnp.maximum(z + b1_ref[...], 0.0)
    t = jnp.dot(z, w2_ref[...], preferred_element_type=jnp.float32)
    t = jnp.maximum(t + b2_ref[...], 0.0)
    t_ref[...] = t
    stats = jnp.concatenate([jnp.sum(t, 0, keepdims=True),
                             jnp.sum(t * t, 0, keepdims=True)], axis=0)

    @pl.when(i == 0)
    def _():
        st_ref[...] = stats

    @pl.when(i > 0)
    def _():
        st_ref[...] += stats


def _tc_mlp_stats(h, a0, a1, w1, b1, w2, b2):
    blk = lambda i: (i, 0)
    const = lambda i: (0, 0)
    return pl.pallas_call(
        _mlp_stats_body,
        grid=(_NB,),
        in_specs=[
            pl.BlockSpec((_BN, _D), blk),
            pl.BlockSpec((_BN, _D), blk),
            pl.BlockSpec((_BN, _D), blk),
            pl.BlockSpec((_D, _D), const),
            pl.BlockSpec((1, _D), const),
            pl.BlockSpec((_D, _D), const),
            pl.BlockSpec((1, _D), const),
        ],
        out_specs=[
            pl.BlockSpec((_BN, _D), blk),
            pl.BlockSpec((2, _D), const),
        ],
        out_shape=[
            jax.ShapeDtypeStruct((_N, _D), jnp.float32),
            jax.ShapeDtypeStruct((2, _D), jnp.float32),
        ],
    )(h, a0, a1, w1, b1, w2, b2)


def _norm_pool_body(t_ref, st_ref, g_ref, b_ref, bt_ref, h_ref, p_ref):
    i = pl.program_id(0)
    mean = st_ref[0:1, :] * (1.0 / _N)
    var = st_ref[1:2, :] * (1.0 / _N) - mean * mean
    scale = lax.rsqrt(var + 1e-5) * g_ref[...]
    off = b_ref[...] - mean * scale
    hh = t_ref[...] * scale + off
    h_ref[...] = hh
    bt = bt_ref[0, :, :]
    gids = lax.broadcasted_iota(jnp.int32, (_G, _BN), 0)
    onehot = (bt == gids).astype(jnp.float32)
    contrib = jnp.dot(onehot, hh, preferred_element_type=jnp.float32)

    @pl.when(i == 0)
    def _():
        p_ref[...] = contrib

    @pl.when(i > 0)
    def _():
        p_ref[...] += contrib


def _tc_norm_pool(t, stats, gamma, beta, batch3):
    blk = lambda i: (i, 0)
    const = lambda i: (0, 0)
    return pl.pallas_call(
        _norm_pool_body,
        grid=(_NB,),
        in_specs=[
            pl.BlockSpec((_BN, _D), blk),
            pl.BlockSpec((2, _D), const),
            pl.BlockSpec((1, _D), const),
            pl.BlockSpec((1, _D), const),
            pl.BlockSpec((1, 1, _BN), lambda i: (i, 0, 0)),
        ],
        out_specs=[
            pl.BlockSpec((_BN, _D), blk),
            pl.BlockSpec((_G, _D), const),
        ],
        out_shape=[
            jax.ShapeDtypeStruct((_N, _D), jnp.float32),
            jax.ShapeDtypeStruct((_G, _D), jnp.float32),
        ],
    )(t, stats, gamma, beta, batch3)


def _head_body(p0_ref, p1_ref, p2_ref, w1_ref, b1_ref, w2_ref, b2_ref,
               yn_ref, xn_ref):
    xc = jnp.concatenate([p0_ref[...], p1_ref[...], p2_ref[...]], axis=1)
    z = jnp.dot(xc, w1_ref[...], preferred_element_type=jnp.float32)
    z = jnp.maximum(z + b1_ref[...], 0.0)
    y = jnp.dot(z, w2_ref[...], preferred_element_type=jnp.float32) + b2_ref[...]
    xnorm = jnp.sqrt(jnp.sum(xc * xc, axis=1, keepdims=True))
    ynorm = jnp.sqrt(jnp.sum(y * y, axis=1, keepdims=True))
    xn_ref[...] = xc / jnp.maximum(xnorm, 1e-12)
    yn_ref[...] = y / jnp.maximum(ynorm, 1e-12)


def _tc_head(p0, p1, p2, pw1, pb1, pw2, pb2):
    H = 3 * _D
    return pl.pallas_call(
        _head_body,
        out_shape=[
            jax.ShapeDtypeStruct((_G, H), jnp.float32),
            jax.ShapeDtypeStruct((_G, H), jnp.float32),
        ],
    )(p0, p1, p2, pw1, pb1, pw2, pb2)


def kernel(x, edge_index, batch,
           l0_W1, l0_b1, l0_W2, l0_b2, l0_gamma, l0_beta,
           l1_W1, l1_b1, l1_W2, l1_b2, l1_gamma, l1_beta,
           l2_W1, l2_b1, l2_W2, l2_b2, l2_gamma, l2_beta,
           p_W1, p_b1, p_W2, p_b2):
    src1 = edge_index[0]
    dst1 = edge_index[1]
    batch3 = batch.reshape(_NB, 1, _BN)
    zeros = jnp.zeros((_NP, _D), jnp.float32)
    layers = [
        (l0_W1, l0_b1, l0_W2, l0_b2, l0_gamma, l0_beta),
        (l1_W1, l1_b1, l1_W2, l1_b2, l1_gamma, l1_beta),
        (l2_W1, l2_b1, l2_W2, l2_b2, l2_gamma, l2_beta),
    ]
    h = x
    pools = []
    for (w1, b1, w2, b2, g, b) in layers:
        a = _sc_segment_sum(h, src1, dst1, zeros)
        t, st = _tc_mlp_stats(h, a[0, :_N], a[1, :_N], w1, b1.reshape(1, _D),
                              w2, b2.reshape(1, _D))
        h, p = _tc_norm_pool(t, st, g.reshape(1, _D), b.reshape(1, _D), batch3)
        pools.append(p)
    yn, xn = _tc_head(pools[0], pools[1], pools[2],
                      p_W1, p_b1.reshape(1, 3 * _D), p_W2, p_b2.reshape(1, 3 * _D))
    return (yn, xn)


# speedup vs baseline: 2.8884x; 1.4998x over previous
import functools

import jax
import jax.numpy as jnp
from jax import lax
from jax.experimental import pallas as pl
from jax.experimental.pallas import tpu as pltpu
from jax.experimental.pallas import tpu_sc as plsc

_N = 10000
_E = 320000
_D = 128
_G = 128
_NB = 10
_BN = _N // _NB

_K = 128
_NC = 2
_NS = 16
_NW = _NC * _NS
_CHUNKS = _E // _K
_FULL = _CHUNKS // _NW
_REM = _CHUNKS - _FULL * _NW
_NP = 10240
_RPT = _NP // _NS


def _sc_segment_sum(h, src1, dst1, zeros):
    mesh = plsc.VectorSubcoreMesh(core_axis_name="c", subcore_axis_name="s")

    @functools.partial(
        pl.kernel,
        out_type=jax.ShapeDtypeStruct((_NC, _NP, _D), jnp.float32),
        mesh=mesh,
        scratch_types=[
            pltpu.VMEM((_K,), jnp.int32),
            pltpu.VMEM((_K,), jnp.int32),
            pltpu.VMEM((_K,), jnp.int32),
            pltpu.VMEM((_K,), jnp.int32),
            pltpu.VMEM((_K, _D), jnp.float32),
            pltpu.VMEM((_K, _D), jnp.float32),
            pltpu.VMEM_SHARED((_NP, _D), jnp.float32),
            pltpu.SemaphoreType.DMA,
            pltpu.SemaphoreType.DMA,
        ],
    )
    def k(h_hbm, src_hbm, dst_hbm, z_hbm, out_hbm, sbuf0, sbuf1, dbuf0, dbuf1,
          rows0, rows1, agg, sem0, sem1):
        c = lax.axis_index("c")
        s = lax.axis_index("s")
        w = s * _NC + c
        sbuf = (sbuf0, sbuf1)
        dbuf = (dbuf0, dbuf1)
        rows = (rows0, rows1)
        sem = (sem0, sem1)

        pltpu.sync_copy(z_hbm.at[pl.ds(s * _RPT, _RPT)],
                        agg.at[pl.ds(s * _RPT, _RPT)])
        plsc.subcore_barrier()

        def load_and_gather(jj, b):
            base = pl.multiple_of((w + jj * _NW) * _K, _K)
            pltpu.sync_copy(src_hbm.at[pl.ds(base, _K)], sbuf[b])
            pltpu.sync_copy(dst_hbm.at[pl.ds(base, _K)], dbuf[b])
            pltpu.async_copy(h_hbm.at[sbuf[b]], rows[b], sem[b])

        load_and_gather(0, 0)

        def body(j, carry):
            for b in range(2):
                jj = j * 2 + b

                @pl.when(jj + 1 < _FULL)
                def _():
                    load_and_gather(jj + 1, 1 - b)

                pltpu.make_async_copy(h_hbm.at[sbuf[b]], rows[b],
                                      sem[b]).wait()
                pltpu.sync_copy(rows[b], agg.at[dbuf[b]], add=True)
            return carry

        lax.fori_loop(0, _FULL // 2, body, 0)

        @pl.when(w < _REM)
        def _():
            base = pl.multiple_of((w + _FULL * _NW) * _K, _K)
            pltpu.sync_copy(src_hbm.at[pl.ds(base, _K)], sbuf[0])
            pltpu.sync_copy(dst_hbm.at[pl.ds(base, _K)], dbuf[0])
            pltpu.async_copy(h_hbm.at[sbuf[0]], rows[0], sem[0]).wait()
            pltpu.sync_copy(rows[0], agg.at[dbuf[0]], add=True)

        plsc.subcore_barrier()
        pltpu.sync_copy(agg.at[pl.ds(s * _RPT, _RPT)],
                        out_hbm.at[c].at[pl.ds(s * _RPT, _RPT)])

    return k(h, src1, dst1, zeros)


def _mlp_stats_body(h_ref, a0_ref, a1_ref, w1_ref, b1_ref, w2_ref, b2_ref,
                    t_ref, st_ref):
    i = pl.program_id(0)
    m = h_ref[...] + a0_ref[...] + a1_ref[...]
    z = jnp.dot(m, w1_ref[...], preferred_element_type=jnp.float32)
    z = jnp.maximum(z + b1_ref[...], 0.0)
    t = jnp.dot(z, w2_ref[...], preferred_element_type=jnp.float32)
    t = jnp.maximum(t + b2_ref[...], 0.0)
    t_ref[...] = t
    stats = jnp.concatenate([jnp.sum(t, 0, keepdims=True),
                             jnp.sum(t * t, 0, keepdims=True)], axis=0)

    @pl.when(i == 0)
    def _():
        st_ref[...] = stats

    @pl.when(i > 0)
    def _():
        st_ref[...] += stats


def _tc_mlp_stats(h, a0, a1, w1, b1, w2, b2):
    blk = lambda i: (i, 0)
    const = lambda i: (0, 0)
    return pl.pallas_call(
        _mlp_stats_body,
        grid=(_NB,),
        in_specs=[
            pl.BlockSpec((_BN, _D), blk),
            pl.BlockSpec((_BN, _D), blk),
            pl.BlockSpec((_BN, _D), blk),
            pl.BlockSpec((_D, _D), const),
            pl.BlockSpec((1, _D), const),
            pl.BlockSpec((_D, _D), const),
            pl.BlockSpec((1, _D), const),
        ],
        out_specs=[
            pl.BlockSpec((_BN, _D), blk),
            pl.BlockSpec((2, _D), const),
        ],
        out_shape=[
            jax.ShapeDtypeStruct((_N, _D), jnp.float32),
            jax.ShapeDtypeStruct((2, _D), jnp.float32),
        ],
    )(h, a0, a1, w1, b1, w2, b2)


def _norm_pool_body(t_ref, st_ref, g_ref, b_ref, bt_ref, h_ref, p_ref):
    i = pl.program_id(0)
    mean = st_ref[0:1, :] * (1.0 / _N)
    var = st_ref[1:2, :] * (1.0 / _N) - mean * mean
    scale = lax.rsqrt(var + 1e-5) * g_ref[...]
    off = b_ref[...] - mean * scale
    hh = t_ref[...] * scale + off
    h_ref[...] = hh
    bt = bt_ref[0, :, :]
    gids = lax.broadcasted_iota(jnp.int32, (_G, _BN), 0)
    onehot = (bt == gids).astype(jnp.float32)
    contrib = jnp.dot(onehot, hh, preferred_element_type=jnp.float32)

    @pl.when(i == 0)
    def _():
        p_ref[...] = contrib

    @pl.when(i > 0)
    def _():
        p_ref[...] += contrib


def _tc_norm_pool(t, stats, gamma, beta, batch3):
    blk = lambda i: (i, 0)
    const = lambda i: (0, 0)
    return pl.pallas_call(
        _norm_pool_body,
        grid=(_NB,),
        in_specs=[
            pl.BlockSpec((_BN, _D), blk),
            pl.BlockSpec((2, _D), const),
            pl.BlockSpec((1, _D), const),
            pl.BlockSpec((1, _D), const),
            pl.BlockSpec((1, 1, _BN), lambda i: (i, 0, 0)),
        ],
        out_specs=[
            pl.BlockSpec((_BN, _D), blk),
            pl.BlockSpec((_G, _D), const),
        ],
        out_shape=[
            jax.ShapeDtypeStruct((_N, _D), jnp.float32),
            jax.ShapeDtypeStruct((_G, _D), jnp.float32),
        ],
    )(t, stats, gamma, beta, batch3)


def _head_body(p0_ref, p1_ref, p2_ref, w1_ref, b1_ref, w2_ref, b2_ref,
               yn_ref, xn_ref):
    xc = jnp.concatenate([p0_ref[...], p1_ref[...], p2_ref[...]], axis=1)
    z = jnp.dot(xc, w1_ref[...], preferred_element_type=jnp.float32)
    z = jnp.maximum(z + b1_ref[...], 0.0)
    y = jnp.dot(z, w2_ref[...], preferred_element_type=jnp.float32) + b2_ref[...]
    xnorm = jnp.sqrt(jnp.sum(xc * xc, axis=1, keepdims=True))
    ynorm = jnp.sqrt(jnp.sum(y * y, axis=1, keepdims=True))
    xn_ref[...] = xc / jnp.maximum(xnorm, 1e-12)
    yn_ref[...] = y / jnp.maximum(ynorm, 1e-12)


def _tc_head(p0, p1, p2, pw1, pb1, pw2, pb2):
    H = 3 * _D
    return pl.pallas_call(
        _head_body,
        out_shape=[
            jax.ShapeDtypeStruct((_G, H), jnp.float32),
            jax.ShapeDtypeStruct((_G, H), jnp.float32),
        ],
    )(p0, p1, p2, pw1, pb1, pw2, pb2)


def kernel(x, edge_index, batch,
           l0_W1, l0_b1, l0_W2, l0_b2, l0_gamma, l0_beta,
           l1_W1, l1_b1, l1_W2, l1_b2, l1_gamma, l1_beta,
           l2_W1, l2_b1, l2_W2, l2_b2, l2_gamma, l2_beta,
           p_W1, p_b1, p_W2, p_b2):
    src1 = edge_index[0]
    dst1 = edge_index[1]
    batch3 = batch.reshape(_NB, 1, _BN)
    zeros = jnp.zeros((_NP, _D), jnp.float32)
    layers = [
        (l0_W1, l0_b1, l0_W2, l0_b2, l0_gamma, l0_beta),
        (l1_W1, l1_b1, l1_W2, l1_b2, l1_gamma, l1_beta),
        (l2_W1, l2_b1, l2_W2, l2_b2, l2_gamma, l2_beta),
    ]
    h = x
    pools = []
    for (w1, b1, w2, b2, g, b) in layers:
        a = _sc_segment_sum(h, src1, dst1, zeros)
        t, st = _tc_mlp_stats(h, a[0, :_N], a[1, :_N], w1, b1.reshape(1, _D),
                              w2, b2.reshape(1, _D))
        h, p = _tc_norm_pool(t, st, g.reshape(1, _D), b.reshape(1, _D), batch3)
        pools.append(p)
    yn, xn = _tc_head(pools[0], pools[1], pools[2],
                      p_W1, p_b1.reshape(1, 3 * _D), p_W2, p_b2.reshape(1, 3 * _D))
    return (yn, xn)


# speedup vs baseline: 3.2236x; 1.1161x over previous
import functools

import jax
import jax.numpy as jnp
from jax import lax
from jax.experimental import pallas as pl
from jax.experimental.pallas import tpu as pltpu
from jax.experimental.pallas import tpu_sc as plsc

_N = 10000
_E = 320000
_D = 128
_G = 128
_NB = 10
_BN = _N // _NB

_K = 128
_NC = 2
_NS = 16
_NW = _NC * _NS
_CHUNKS = _E // _K
_FULL = _CHUNKS // _NW
_REM = _CHUNKS - _FULL * _NW
_NP = 10240
_RPT = _NP // _NS


def _sc_segment_sum(h, src1, dst1, zeros):
    mesh = plsc.VectorSubcoreMesh(core_axis_name="c", subcore_axis_name="s")

    @functools.partial(
        pl.kernel,
        out_type=jax.ShapeDtypeStruct((_NC, _NP, _D), jnp.float32),
        mesh=mesh,
        scratch_types=[
            pltpu.VMEM((_K,), jnp.int32),
            pltpu.VMEM((_K,), jnp.int32),
            pltpu.VMEM((_K,), jnp.int32),
            pltpu.VMEM((_K,), jnp.int32),
            pltpu.VMEM((_K, _D), jnp.float32),
            pltpu.VMEM((_K, _D), jnp.float32),
            pltpu.VMEM_SHARED((_NP, _D), jnp.float32),
            pltpu.SemaphoreType.DMA,
            pltpu.SemaphoreType.DMA,
            pltpu.SemaphoreType.DMA,
            pltpu.SemaphoreType.DMA,
        ],
    )
    def k(h_hbm, src_hbm, dst_hbm, z_hbm, out_hbm, sbuf0, sbuf1, dbuf0, dbuf1,
          rows0, rows1, agg, semi0, semi1, semg0, semg1):
        c = lax.axis_index("c")
        s = lax.axis_index("s")
        w = s * _NC + c
        sbuf = (sbuf0, sbuf1)
        dbuf = (dbuf0, dbuf1)
        rows = (rows0, rows1)
        semi = (semi0, semi1)
        semg = (semg0, semg1)

        def eoff(jj):
            return pl.multiple_of((w + jj * _NW) * _K, _K)

        def issue_idx(jj, b):
            pltpu.async_copy(src_hbm.at[pl.ds(eoff(jj), _K)], sbuf[b], semi[b])
            pltpu.async_copy(dst_hbm.at[pl.ds(eoff(jj), _K)], dbuf[b], semi[b])

        def wait_idx(jj, b):
            pltpu.make_async_copy(src_hbm.at[pl.ds(eoff(jj), _K)], sbuf[b],
                                  semi[b]).wait()
            pltpu.make_async_copy(dst_hbm.at[pl.ds(eoff(jj), _K)], dbuf[b],
                                  semi[b]).wait()

        issue_idx(0, 0)
        issue_idx(1, 1)
        pltpu.sync_copy(z_hbm.at[pl.ds(s * _RPT, _RPT)],
                        agg.at[pl.ds(s * _RPT, _RPT)])
        plsc.subcore_barrier()
        wait_idx(0, 0)
        pltpu.async_copy(h_hbm.at[sbuf[0]], rows[0], semg[0])

        def body(j, carry):
            for b in range(2):
                jj = j * 2 + b
                pltpu.make_async_copy(h_hbm.at[sbuf[b]], rows[b],
                                      semg[b]).wait()

                @pl.when(jj + 1 < _FULL)
                def _():
                    wait_idx(jj + 1, 1 - b)
                    pltpu.async_copy(h_hbm.at[sbuf[1 - b]], rows[1 - b],
                                     semg[1 - b])

                pltpu.sync_copy(rows[b], agg.at[dbuf[b]], add=True)

                @pl.when(jj + 2 < _FULL)
                def _():
                    issue_idx(jj + 2, b)
            return carry

        lax.fori_loop(0, _FULL // 2, body, 0)

        @pl.when(w < _REM)
        def _():
            base = pl.multiple_of((w + _FULL * _NW) * _K, _K)
            pltpu.sync_copy(src_hbm.at[pl.ds(base, _K)], sbuf[0])
            pltpu.sync_copy(dst_hbm.at[pl.ds(base, _K)], dbuf[0])
            pltpu.async_copy(h_hbm.at[sbuf[0]], rows[0], semg[0]).wait()
            pltpu.sync_copy(rows[0], agg.at[dbuf[0]], add=True)

        plsc.subcore_barrier()
        pltpu.sync_copy(agg.at[pl.ds(s * _RPT, _RPT)],
                        out_hbm.at[c].at[pl.ds(s * _RPT, _RPT)])

    return k(h, src1, dst1, zeros)


def _mlp_stats_body(h_ref, a0_ref, a1_ref, w1_ref, b1_ref, w2_ref, b2_ref,
                    t_ref, st_ref):
    i = pl.program_id(0)
    m = h_ref[...] + a0_ref[...] + a1_ref[...]
    z = jnp.dot(m, w1_ref[...], preferred_element_type=jnp.float32)
    z = jnp.maximum(z + b1_ref[...], 0.0)
    t = jnp.dot(z, w2_ref[...], preferred_element_type=jnp.float32)
    t = jnp.maximum(t + b2_ref[...], 0.0)
    t_ref[...] = t
    stats = jnp.concatenate([jnp.sum(t, 0, keepdims=True),
                             jnp.sum(t * t, 0, keepdims=True)], axis=0)

    @pl.when(i == 0)
    def _():
        st_ref[...] = stats

    @pl.when(i > 0)
    def _():
        st_ref[...] += stats


def _tc_mlp_stats(h, a0, a1, w1, b1, w2, b2):
    blk = lambda i: (i, 0)
    const = lambda i: (0, 0)
    return pl.pallas_call(
        _mlp_stats_body,
        grid=(_NB,),
        in_specs=[
            pl.BlockSpec((_BN, _D), blk),
            pl.BlockSpec((_BN, _D), blk),
            pl.BlockSpec((_BN, _D), blk),
            pl.BlockSpec((_D, _D), const),
            pl.BlockSpec((1, _D), const),
            pl.BlockSpec((_D, _D), const),
            pl.BlockSpec((1, _D), const),
        ],
        out_specs=[
            pl.BlockSpec((_BN, _D), blk),
            pl.BlockSpec((2, _D), const),
        ],
        out_shape=[
            jax.ShapeDtypeStruct((_N, _D), jnp.float32),
            jax.ShapeDtypeStruct((2, _D), jnp.float32),
        ],
    )(h, a0, a1, w1, b1, w2, b2)


def _norm_pool_body(t_ref, st_ref, g_ref, b_ref, bt_ref, h_ref, p_ref):
    i = pl.program_id(0)
    mean = st_ref[0:1, :] * (1.0 / _N)
    var = st_ref[1:2, :] * (1.0 / _N) - mean * mean
    scale = lax.rsqrt(var + 1e-5) * g_ref[...]
    off = b_ref[...] - mean * scale
    hh = t_ref[...] * scale + off
    h_ref[...] = hh
    bt = bt_ref[0, :, :]
    gids = lax.broadcasted_iota(jnp.int32, (_G, _BN), 0)
    onehot = (bt == gids).astype(jnp.float32)
    contrib = jnp.dot(onehot, hh, preferred_element_type=jnp.float32)

    @pl.when(i == 0)
    def _():
        p_ref[...] = contrib

    @pl.when(i > 0)
    def _():
        p_ref[...] += contrib


def _tc_norm_pool(t, stats, gamma, beta, batch3):
    blk = lambda i: (i, 0)
    const = lambda i: (0, 0)
    return pl.pallas_call(
        _norm_pool_body,
        grid=(_NB,),
        in_specs=[
            pl.BlockSpec((_BN, _D), blk),
            pl.BlockSpec((2, _D), const),
            pl.BlockSpec((1, _D), const),
            pl.BlockSpec((1, _D), const),
            pl.BlockSpec((1, 1, _BN), lambda i: (i, 0, 0)),
        ],
        out_specs=[
            pl.BlockSpec((_BN, _D), blk),
            pl.BlockSpec((_G, _D), const),
        ],
        out_shape=[
            jax.ShapeDtypeStruct((_N, _D), jnp.float32),
            jax.ShapeDtypeStruct((_G, _D), jnp.float32),
        ],
    )(t, stats, gamma, beta, batch3)


def _head_body(p0_ref, p1_ref, p2_ref, w1_ref, b1_ref, w2_ref, b2_ref,
               yn_ref, xn_ref):
    xc = jnp.concatenate([p0_ref[...], p1_ref[...], p2_ref[...]], axis=1)
    z = jnp.dot(xc, w1_ref[...], preferred_element_type=jnp.float32)
    z = jnp.maximum(z + b1_ref[...], 0.0)
    y = jnp.dot(z, w2_ref[...], preferred_element_type=jnp.float32) + b2_ref[...]
    xnorm = jnp.sqrt(jnp.sum(xc * xc, axis=1, keepdims=True))
    ynorm = jnp.sqrt(jnp.sum(y * y, axis=1, keepdims=True))
    xn_ref[...] = xc / jnp.maximum(xnorm, 1e-12)
    yn_ref[...] = y / jnp.maximum(ynorm, 1e-12)


def _tc_head(p0, p1, p2, pw1, pb1, pw2, pb2):
    H = 3 * _D
    return pl.pallas_call(
        _head_body,
        out_shape=[
            jax.ShapeDtypeStruct((_G, H), jnp.float32),
            jax.ShapeDtypeStruct((_G, H), jnp.float32),
        ],
    )(p0, p1, p2, pw1, pb1, pw2, pb2)


def kernel(x, edge_index, batch,
           l0_W1, l0_b1, l0_W2, l0_b2, l0_gamma, l0_beta,
           l1_W1, l1_b1, l1_W2, l1_b2, l1_gamma, l1_beta,
           l2_W1, l2_b1, l2_W2, l2_b2, l2_gamma, l2_beta,
           p_W1, p_b1, p_W2, p_b2):
    src1 = edge_index[0]
    dst1 = edge_index[1]
    batch3 = batch.reshape(_NB, 1, _BN)
    zeros = jnp.zeros((_NP, _D), jnp.float32)
    layers = [
        (l0_W1, l0_b1, l0_W2, l0_b2, l0_gamma, l0_beta),
        (l1_W1, l1_b1, l1_W2, l1_b2, l1_gamma, l1_beta),
        (l2_W1, l2_b1, l2_W2, l2_b2, l2_gamma, l2_beta),
    ]
    h = x
    pools = []
    for (w1, b1, w2, b2, g, b) in layers:
        a = _sc_segment_sum(h, src1, dst1, zeros)
        t, st = _tc_mlp_stats(h, a[0, :_N], a[1, :_N], w1, b1.reshape(1, _D),
                              w2, b2.reshape(1, _D))
        h, p = _tc_norm_pool(t, st, g.reshape(1, _D), b.reshape(1, _D), batch3)
        pools.append(p)
    yn, xn = _tc_head(pools[0], pools[1], pools[2],
                      p_W1, p_b1.reshape(1, 3 * _D), p_W2, p_b2.reshape(1, 3 * _D))
    return (yn, xn)


# speedup vs baseline: 3.3492x; 1.0390x over previous
import functools

import jax
import jax.numpy as jnp
from jax import lax
from jax.experimental import pallas as pl
from jax.experimental.pallas import tpu as pltpu
from jax.experimental.pallas import tpu_sc as plsc

_N = 10000
_E = 320000
_D = 128
_G = 128
_NB = 10
_BN = _N // _NB

_K = 128
_NC = 2
_NS = 16
_NW = _NC * _NS
_CHUNKS = _E // _K
_FULL = _CHUNKS // _NW
_REM = _CHUNKS - _FULL * _NW
_NP = 10240
_RPT = _NP // _NS


def _sc_segment_sum(h, src1, dst1, zeros):
    mesh = plsc.VectorSubcoreMesh(core_axis_name="c", subcore_axis_name="s")

    @functools.partial(
        pl.kernel,
        out_type=jax.ShapeDtypeStruct((_NC, _NP, _D), jnp.float32),
        mesh=mesh,
        scratch_types=[
            pltpu.VMEM((_K,), jnp.int32),
            pltpu.VMEM((_K,), jnp.int32),
            pltpu.VMEM((_K,), jnp.int32),
            pltpu.VMEM((_K,), jnp.int32),
            pltpu.VMEM((_K, _D), jnp.float32),
            pltpu.VMEM((_K, _D), jnp.float32),
            pltpu.VMEM_SHARED((_NP, _D), jnp.float32),
            pltpu.SemaphoreType.DMA,
            pltpu.SemaphoreType.DMA,
            pltpu.SemaphoreType.DMA,
            pltpu.SemaphoreType.DMA,
        ],
    )
    def k(h_hbm, src_hbm, dst_hbm, z_hbm, out_hbm, sbuf0, sbuf1, dbuf0, dbuf1,
          rows0, rows1, agg, semi0, semi1, semg0, semg1):
        c = lax.axis_index("c")
        s = lax.axis_index("s")
        w = s * _NC + c
        sbuf = (sbuf0, sbuf1)
        dbuf = (dbuf0, dbuf1)
        rows = (rows0, rows1)
        semi = (semi0, semi1)
        semg = (semg0, semg1)

        def eoff(jj):
            return pl.multiple_of((w + jj * _NW) * _K, _K)

        def issue_idx(jj, b):
            pltpu.async_copy(src_hbm.at[pl.ds(eoff(jj), _K)], sbuf[b], semi[b])
            pltpu.async_copy(dst_hbm.at[pl.ds(eoff(jj), _K)], dbuf[b], semi[b])

        def wait_idx(jj, b):
            pltpu.make_async_copy(src_hbm.at[pl.ds(eoff(jj), _K)], sbuf[b],
                                  semi[b]).wait()
            pltpu.make_async_copy(dst_hbm.at[pl.ds(eoff(jj), _K)], dbuf[b],
                                  semi[b]).wait()

        issue_idx(0, 0)
        issue_idx(1, 1)
        pltpu.sync_copy(z_hbm.at[pl.ds(s * _RPT, _RPT)],
                        agg.at[pl.ds(s * _RPT, _RPT)])
        plsc.subcore_barrier()
        wait_idx(0, 0)
        pltpu.async_copy(h_hbm.at[sbuf[0]], rows[0], semg[0])

        def body(j, carry):
            for b in range(2):
                jj = j * 2 + b

                @pl.when(jj + 1 < _FULL)
                def _():
                    wait_idx(jj + 1, 1 - b)
                    pltpu.async_copy(h_hbm.at[sbuf[1 - b]], rows[1 - b],
                                     semg[1 - b])

                pltpu.make_async_copy(h_hbm.at[sbuf[b]], rows[b],
                                      semg[b]).wait()
                pltpu.sync_copy(rows[b], agg.at[dbuf[b]], add=True)

                @pl.when(jj + 2 < _FULL)
                def _():
                    issue_idx(jj + 2, b)
            return carry

        lax.fori_loop(0, _FULL // 2, body, 0)

        @pl.when(w < _REM)
        def _():
            base = pl.multiple_of((w + _FULL * _NW) * _K, _K)
            pltpu.sync_copy(src_hbm.at[pl.ds(base, _K)], sbuf[0])
            pltpu.sync_copy(dst_hbm.at[pl.ds(base, _K)], dbuf[0])
            pltpu.async_copy(h_hbm.at[sbuf[0]], rows[0], semg[0]).wait()
            pltpu.sync_copy(rows[0], agg.at[dbuf[0]], add=True)

        plsc.subcore_barrier()
        pltpu.sync_copy(agg.at[pl.ds(s * _RPT, _RPT)],
                        out_hbm.at[c].at[pl.ds(s * _RPT, _RPT)])

    return k(h, src1, dst1, zeros)


def _mlp_stats_body(h_ref, a0_ref, a1_ref, w1_ref, b1_ref, w2_ref, b2_ref,
                    t_ref, st_ref):
    i = pl.program_id(0)
    m = h_ref[...] + a0_ref[...] + a1_ref[...]
    z = jnp.dot(m, w1_ref[...], preferred_element_type=jnp.float32)
    z = jnp.maximum(z + b1_ref[...], 0.0)
    t = jnp.dot(z, w2_ref[...], preferred_element_type=jnp.float32)
    t = jnp.maximum(t + b2_ref[...], 0.0)
    t_ref[...] = t
    stats = jnp.concatenate([jnp.sum(t, 0, keepdims=True),
                             jnp.sum(t * t, 0, keepdims=True)], axis=0)

    @pl.when(i == 0)
    def _():
        st_ref[...] = stats

    @pl.when(i > 0)
    def _():
        st_ref[...] += stats


def _tc_mlp_stats(h, a0, a1, w1, b1, w2, b2):
    blk = lambda i: (i, 0)
    const = lambda i: (0, 0)
    return pl.pallas_call(
        _mlp_stats_body,
        grid=(_NB,),
        in_specs=[
            pl.BlockSpec((_BN, _D), blk),
            pl.BlockSpec((_BN, _D), blk),
            pl.BlockSpec((_BN, _D), blk),
            pl.BlockSpec((_D, _D), const),
            pl.BlockSpec((1, _D), const),
            pl.BlockSpec((_D, _D), const),
            pl.BlockSpec((1, _D), const),
        ],
        out_specs=[
            pl.BlockSpec((_BN, _D), blk),
            pl.BlockSpec((2, _D), const),
        ],
        out_shape=[
            jax.ShapeDtypeStruct((_N, _D), jnp.float32),
            jax.ShapeDtypeStruct((2, _D), jnp.float32),
        ],
    )(h, a0, a1, w1, b1, w2, b2)


def _norm_pool_body(t_ref, st_ref, g_ref, b_ref, bt_ref, h_ref, p_ref):
    i = pl.program_id(0)
    mean = st_ref[0:1, :] * (1.0 / _N)
    var = st_ref[1:2, :] * (1.0 / _N) - mean * mean
    scale = lax.rsqrt(var + 1e-5) * g_ref[...]
    off = b_ref[...] - mean * scale
    hh = t_ref[...] * scale + off
    h_ref[...] = hh
    bt = bt_ref[0, :, :]
    gids = lax.broadcasted_iota(jnp.int32, (_G, _BN), 0)
    onehot = (bt == gids).astype(jnp.float32)
    contrib = jnp.dot(onehot, hh, preferred_element_type=jnp.float32)

    @pl.when(i == 0)
    def _():
        p_ref[...] = contrib

    @pl.when(i > 0)
    def _():
        p_ref[...] += contrib


def _tc_norm_pool(t, stats, gamma, beta, batch3):
    blk = lambda i: (i, 0)
    const = lambda i: (0, 0)
    return pl.pallas_call(
        _norm_pool_body,
        grid=(_NB,),
        in_specs=[
            pl.BlockSpec((_BN, _D), blk),
            pl.BlockSpec((2, _D), const),
            pl.BlockSpec((1, _D), const),
            pl.BlockSpec((1, _D), const),
            pl.BlockSpec((1, 1, _BN), lambda i: (i, 0, 0)),
        ],
        out_specs=[
            pl.BlockSpec((_BN, _D), blk),
            pl.BlockSpec((_G, _D), const),
        ],
        out_shape=[
            jax.ShapeDtypeStruct((_N, _D), jnp.float32),
            jax.ShapeDtypeStruct((_G, _D), jnp.float32),
        ],
    )(t, stats, gamma, beta, batch3)


def _head_body(p0_ref, p1_ref, p2_ref, w1_ref, b1_ref, w2_ref, b2_ref,
               yn_ref, xn_ref):
    xc = jnp.concatenate([p0_ref[...], p1_ref[...], p2_ref[...]], axis=1)
    z = jnp.dot(xc, w1_ref[...], preferred_element_type=jnp.float32)
    z = jnp.maximum(z + b1_ref[...], 0.0)
    y = jnp.dot(z, w2_ref[...], preferred_element_type=jnp.float32) + b2_ref[...]
    xnorm = jnp.sqrt(jnp.sum(xc * xc, axis=1, keepdims=True))
    ynorm = jnp.sqrt(jnp.sum(y * y, axis=1, keepdims=True))
    xn_ref[...] = xc / jnp.maximum(xnorm, 1e-12)
    yn_ref[...] = y / jnp.maximum(ynorm, 1e-12)


def _tc_head(p0, p1, p2, pw1, pb1, pw2, pb2):
    H = 3 * _D
    return pl.pallas_call(
        _head_body,
        out_shape=[
            jax.ShapeDtypeStruct((_G, H), jnp.float32),
            jax.ShapeDtypeStruct((_G, H), jnp.float32),
        ],
    )(p0, p1, p2, pw1, pb1, pw2, pb2)


def kernel(x, edge_index, batch,
           l0_W1, l0_b1, l0_W2, l0_b2, l0_gamma, l0_beta,
           l1_W1, l1_b1, l1_W2, l1_b2, l1_gamma, l1_beta,
           l2_W1, l2_b1, l2_W2, l2_b2, l2_gamma, l2_beta,
           p_W1, p_b1, p_W2, p_b2):
    src1 = edge_index[0]
    dst1 = edge_index[1]
    batch3 = batch.reshape(_NB, 1, _BN)
    zeros = jnp.zeros((_NP, _D), jnp.float32)
    layers = [
        (l0_W1, l0_b1, l0_W2, l0_b2, l0_gamma, l0_beta),
        (l1_W1, l1_b1, l1_W2, l1_b2, l1_gamma, l1_beta),
        (l2_W1, l2_b1, l2_W2, l2_b2, l2_gamma, l2_beta),
    ]
    h = x
    pools = []
    for (w1, b1, w2, b2, g, b) in layers:
        a = _sc_segment_sum(h, src1, dst1, zeros)
        t, st = _tc_mlp_stats(h, a[0, :_N], a[1, :_N], w1, b1.reshape(1, _D),
                              w2, b2.reshape(1, _D))
        h, p = _tc_norm_pool(t, st, g.reshape(1, _D), b.reshape(1, _D), batch3)
        pools.append(p)
    yn, xn = _tc_head(pools[0], pools[1], pools[2],
                      p_W1, p_b1.reshape(1, 3 * _D), p_W2, p_b2.reshape(1, 3 * _D))
    return (yn, xn)
